# baseline (device time: 31352 ns/iter reference)
import jax
import jax.numpy as jnp
from jax import lax
from jax.experimental import pallas as pl
from jax.experimental.pallas import tpu as pltpu

T = 256
D = 512
V = 4096
H = T // 2
CH = 2
R = H // CH
PV = V + 128


def kernel(x, W):
    assert x.shape == (T, D), x.shape
    assert W.shape == (D, V), W.shape

    def body(x_hbm, w_hbm, out_ref, x_ref, w_ref, xsend, xrecv,
             yrecv, eloc_ref, sloc_ref, in_sems,
             xs_sems, xr_sems, fs_sems, fr_sems):
        my_x = lax.axis_index("x")
        my_y = lax.axis_index("y")
        my_z = lax.axis_index("z")
        x_partner = (1 - my_x, my_y, my_z)
        y_partner = (my_x, 1 - my_y, my_z)
        loc_off = my_x * V
        rem_off = (1 - my_x) * V
        pull_base = my_y * H
        other_base = (1 - my_y) * H

        w_cp = pltpu.make_async_copy(w_hbm, w_ref, in_sems.at[0])
        x_cp = pltpu.make_async_copy(x_hbm, x_ref, in_sems.at[1])
        w_cp.start()
        x_cp.start()

        barrier = pltpu.get_barrier_semaphore()
        for nbr in (x_partner, y_partner):
            pl.semaphore_signal(barrier, inc=1, device_id=nbr,
                                device_id_type=pl.DeviceIdType.MESH)
        pl.semaphore_wait(barrier, 2)

        w_cp.wait()
        x_cp.wait()
        w_bf = w_ref[:, :].astype(jnp.bfloat16)

        def local_chunk(base, i):
            rows = pl.ds(base + i * R, R)
            logits = jnp.dot(x_ref[rows, :].astype(jnp.bfloat16), w_bf,
                             preferred_element_type=jnp.float32)
            e = jnp.exp(logits)
            s = jnp.sum(e, axis=-1, keepdims=True)
            return rows, e, s

        xr_rdmas = []
        for i in range(CH):
            rows, e, s = local_chunk(pull_base, i)
            xsend[i] = jnp.concatenate(
                [e.astype(jnp.bfloat16),
                 jnp.broadcast_to(s.astype(jnp.bfloat16), (R, 128))],
                axis=1,
            )
            rdma = pltpu.make_async_remote_copy(
                src_ref=xsend.at[i], dst_ref=xrecv.at[i],
                send_sem=xs_sems.at[i], recv_sem=xr_sems.at[i],
                device_id=x_partner, device_id_type=pl.DeviceIdType.MESH)
            rdma.start()
            xr_rdmas.append(rdma)
            eloc_ref[rows, :] = e
            sloc_ref[rows, :] = s

        for i in range(CH):
            rows, e, s = local_chunk(other_base, i)
            eloc_ref[rows, :] = e
            sloc_ref[rows, :] = s

        def finish_chunk(base, i, buf):
            rows = pl.ds(base + i * R, R)
            blk = buf[i]
            e_rem = blk[:, :V].astype(jnp.float32)
            s_rem = blk[:, V:V + 128].astype(jnp.float32)[:, 0:1]
            inv = 1.0 / (sloc_ref[rows, :] + s_rem)
            out_ref[rows, pl.ds(loc_off, V)] = eloc_ref[rows, :] * inv
            out_ref[rows, pl.ds(rem_off, V)] = e_rem * inv

        fwd_rdmas = []
        for i in range(CH):
            xr_rdmas[i].wait_recv()
            fwd = pltpu.make_async_remote_copy(
                src_ref=xrecv.at[i], dst_ref=yrecv.at[i],
                send_sem=fs_sems.at[i], recv_sem=fr_sems.at[i],
                device_id=y_partner, device_id_type=pl.DeviceIdType.MESH)
            fwd.start()
            fwd_rdmas.append(fwd)
            finish_chunk(pull_base, i, xrecv)

        for i in range(CH):
            fwd_rdmas[i].wait_recv()
            finish_chunk(other_base, i, yrecv)

        for i in range(CH):
            xr_rdmas[i].wait_send()
            fwd_rdmas[i].wait_send()

    return pl.pallas_call(
        body,
        out_shape=jax.ShapeDtypeStruct((T, 2 * V), jnp.float32),
        in_specs=[
            pl.BlockSpec(memory_space=pltpu.MemorySpace.HBM),
            pl.BlockSpec(memory_space=pltpu.MemorySpace.HBM),
        ],
        out_specs=pl.BlockSpec(memory_space=pltpu.VMEM),
        scratch_shapes=[
            pltpu.VMEM((T, D), jnp.float32),
            pltpu.VMEM((D, V), jnp.float32),
            pltpu.VMEM((CH, R, PV), jnp.bfloat16),
            pltpu.VMEM((CH, R, PV), jnp.bfloat16),
            pltpu.VMEM((CH, R, PV), jnp.bfloat16),
            pltpu.VMEM((T, V), jnp.float32),
            pltpu.VMEM((T, 1), jnp.float32),
            pltpu.SemaphoreType.DMA((2,)),
            pltpu.SemaphoreType.DMA((CH,)),
            pltpu.SemaphoreType.DMA((CH,)),
            pltpu.SemaphoreType.DMA((CH,)),
            pltpu.SemaphoreType.DMA((CH,)),
        ],
        compiler_params=pltpu.CompilerParams(collective_id=0),
    )(
        pltpu.with_memory_space_constraint(x, pltpu.MemorySpace.HBM),
        pltpu.with_memory_space_constraint(W, pltpu.MemorySpace.HBM),
    )


# device time: 29201 ns/iter; 1.0737x vs baseline; 1.0737x over previous
import jax
import jax.numpy as jnp
from jax import lax
from jax.experimental import pallas as pl
from jax.experimental.pallas import tpu as pltpu

T = 256
D = 512
V = 4096
H = T // 2
CH = 4
R = H // CH
PV = V + 128


def kernel(x, W):
    assert x.shape == (T, D), x.shape
    assert W.shape == (D, V), W.shape

    def body(x_hbm, w_hbm, out_ref, x_ref, w_ref, xsend, xrecv,
             yrecv, eloc_ref, sloc_ref, in_sems,
             xs_sems, xr_sems, fs_sems, fr_sems):
        my_x = lax.axis_index("x")
        my_y = lax.axis_index("y")
        my_z = lax.axis_index("z")
        x_partner = (1 - my_x, my_y, my_z)
        y_partner = (my_x, 1 - my_y, my_z)
        loc_off = my_x * V
        rem_off = (1 - my_x) * V
        pull_base = my_y * H
        other_base = (1 - my_y) * H

        w_cp = pltpu.make_async_copy(w_hbm, w_ref, in_sems.at[0])
        x_cp = pltpu.make_async_copy(x_hbm, x_ref, in_sems.at[1])
        w_cp.start()
        x_cp.start()

        barrier = pltpu.get_barrier_semaphore()
        for nbr in (x_partner, y_partner):
            pl.semaphore_signal(barrier, inc=1, device_id=nbr,
                                device_id_type=pl.DeviceIdType.MESH)
        pl.semaphore_wait(barrier, 2)

        w_cp.wait()
        x_cp.wait()
        w_bf = w_ref[:, :].astype(jnp.bfloat16)

        def local_chunk(base, i):
            rows = pl.ds(base + i * R, R)
            logits = jnp.dot(x_ref[rows, :].astype(jnp.bfloat16), w_bf,
                             preferred_element_type=jnp.float32)
            e = jnp.exp(logits)
            s = jnp.sum(e, axis=-1, keepdims=True)
            return rows, e, s

        xr_rdmas = []
        for i in range(CH):
            rows, e, s = local_chunk(pull_base, i)
            xsend[i] = jnp.concatenate(
                [e.astype(jnp.bfloat16),
                 jnp.broadcast_to(s.astype(jnp.bfloat16), (R, 128))],
                axis=1,
            )
            rdma = pltpu.make_async_remote_copy(
                src_ref=xsend.at[i], dst_ref=xrecv.at[i],
                send_sem=xs_sems.at[i], recv_sem=xr_sems.at[i],
                device_id=x_partner, device_id_type=pl.DeviceIdType.MESH)
            rdma.start()
            xr_rdmas.append(rdma)
            eloc_ref[rows, :] = e
            sloc_ref[rows, :] = s

        for i in range(CH):
            rows, e, s = local_chunk(other_base, i)
            eloc_ref[rows, :] = e
            sloc_ref[rows, :] = s

        def finish_chunk(base, i, buf):
            rows = pl.ds(base + i * R, R)
            blk = buf[i]
            e_rem = blk[:, :V].astype(jnp.float32)
            s_rem = blk[:, V:V + 128].astype(jnp.float32)[:, 0:1]
            inv = 1.0 / (sloc_ref[rows, :] + s_rem)
            out_ref[rows, pl.ds(loc_off, V)] = eloc_ref[rows, :] * inv
            out_ref[rows, pl.ds(rem_off, V)] = e_rem * inv

        fwd_rdmas = []
        for i in range(CH):
            xr_rdmas[i].wait_recv()
            fwd = pltpu.make_async_remote_copy(
                src_ref=xrecv.at[i], dst_ref=yrecv.at[i],
                send_sem=fs_sems.at[i], recv_sem=fr_sems.at[i],
                device_id=y_partner, device_id_type=pl.DeviceIdType.MESH)
            fwd.start()
            fwd_rdmas.append(fwd)
            finish_chunk(pull_base, i, xrecv)

        for i in range(CH):
            fwd_rdmas[i].wait_recv()
            finish_chunk(other_base, i, yrecv)

        for i in range(CH):
            xr_rdmas[i].wait_send()
            fwd_rdmas[i].wait_send()

    return pl.pallas_call(
        body,
        out_shape=jax.ShapeDtypeStruct((T, 2 * V), jnp.float32),
        in_specs=[
            pl.BlockSpec(memory_space=pltpu.MemorySpace.HBM),
            pl.BlockSpec(memory_space=pltpu.MemorySpace.HBM),
        ],
        out_specs=pl.BlockSpec(memory_space=pltpu.VMEM),
        scratch_shapes=[
            pltpu.VMEM((T, D), jnp.float32),
            pltpu.VMEM((D, V), jnp.float32),
            pltpu.VMEM((CH, R, PV), jnp.bfloat16),
            pltpu.VMEM((CH, R, PV), jnp.bfloat16),
            pltpu.VMEM((CH, R, PV), jnp.bfloat16),
            pltpu.VMEM((T, V), jnp.float32),
            pltpu.VMEM((T, 1), jnp.float32),
            pltpu.SemaphoreType.DMA((2,)),
            pltpu.SemaphoreType.DMA((CH,)),
            pltpu.SemaphoreType.DMA((CH,)),
            pltpu.SemaphoreType.DMA((CH,)),
            pltpu.SemaphoreType.DMA((CH,)),
        ],
        compiler_params=pltpu.CompilerParams(collective_id=0),
    )(
        pltpu.with_memory_space_constraint(x, pltpu.MemorySpace.HBM),
        pltpu.with_memory_space_constraint(W, pltpu.MemorySpace.HBM),
    )
